# baseline (speedup 1.0000x reference)
import jax
import jax.numpy as jnp
from jax.experimental import pallas as pl
from jax.experimental.pallas import tpu as pltpu


def _moe_body(ids_ref, w_ref, x_ref, w13_ref, w2a_ref, w2b_ref, out_ref,
              gate_ref):
    e = pl.program_id(0)
    j = pl.program_id(1)

    @pl.when((e == 0) & (j == 0))
    def _init():
        out_ref[...] = jnp.zeros_like(out_ref)

    x = x_ref[...]
    w13 = w13_ref[0, 0].astype(jnp.bfloat16)

    @pl.when(j == 0)
    def _gate():
        gate_ref[...] = jax.lax.dot_general(
            x, w13, (((1,), (1,)), ((), ())),
            preferred_element_type=jnp.float32)

    @pl.when(j == 1)
    def _up_down():
        ids = ids_ref[...]
        wts = w_ref[...]
        coef = jnp.sum(wts * (ids == e).astype(jnp.float32), axis=1)
        up = jax.lax.dot_general(
            x, w13, (((1,), (1,)), ((), ())),
            preferred_element_type=jnp.float32)
        g = gate_ref[...]
        h = (g * jax.nn.sigmoid(g) * up).astype(jnp.bfloat16)
        o0 = jax.lax.dot_general(
            h, w2a_ref[0], (((1,), (1,)), ((), ())),
            preferred_element_type=jnp.float32)
        o1 = jax.lax.dot_general(
            h, w2b_ref[0], (((1,), (1,)), ((), ())),
            preferred_element_type=jnp.float32)
        c = coef[:, None]
        half = o0.shape[1]
        out_ref[:, :half] += c * o0
        out_ref[:, half:] += c * o1


def kernel(hidden_states, topk_ids, topk_weights, w13, w2):
    tokens, hidden = hidden_states.shape
    num_experts, two_inter, _ = w13.shape
    inter = w2.shape[2]
    topk_ids = topk_ids.astype(jnp.int32)
    topk_weights = topk_weights.astype(jnp.float32)
    x16 = hidden_states.astype(jnp.bfloat16)
    w13r = w13.reshape(num_experts, 2, inter, hidden)

    out = pl.pallas_call(
        _moe_body,
        grid=(num_experts, 2),
        in_specs=[
            pl.BlockSpec(topk_ids.shape, lambda e, j: (0, 0)),
            pl.BlockSpec(topk_weights.shape, lambda e, j: (0, 0)),
            pl.BlockSpec((tokens, hidden), lambda e, j: (0, 0)),
            pl.BlockSpec((1, 1, inter, hidden), lambda e, j: (e, j, 0, 0)),
            pl.BlockSpec((1, hidden // 2, inter), lambda e, j: (e, 0, 0)),
            pl.BlockSpec((1, hidden // 2, inter),
                         lambda e, j: (jnp.maximum(e + j - 1, 0), 1, 0)),
        ],
        out_specs=pl.BlockSpec((tokens, hidden), lambda e, j: (0, 0)),
        out_shape=jax.ShapeDtypeStruct((tokens, hidden), jnp.float32),
        scratch_shapes=[pltpu.VMEM((tokens, inter), jnp.float32)],
    )(topk_ids, topk_weights, x16, w13r, w2, w2)
    return out


# speedup vs baseline: 1.2663x; 1.2663x over previous
import jax
import jax.numpy as jnp
from jax.experimental import pallas as pl
from jax.experimental.pallas import tpu as pltpu

_NP = 768
_CHUNK = 32
_MAX_CHUNKS = 8


def _moe_body(start_ref, nch_ref, g_ref, p_ref, x_ref, w13_ref, w2_ref,
              out_ref, xs_ref, os_ref):
    e = pl.program_id(0)
    nexp = pl.num_programs(0)

    @pl.when(e == 0)
    def _init():
        os_ref[...] = jnp.zeros_like(os_ref)
        xs_ref[...] = jax.lax.dot_general(
            g_ref[...], x_ref[...], (((1,), (0,)), ((), ())),
            preferred_element_type=jnp.float32).astype(jnp.bfloat16)

    w13 = w13_ref[0].astype(jnp.bfloat16)
    w2 = w2_ref[0].astype(jnp.bfloat16)
    inter = w2.shape[1]
    base = start_ref[e]

    for c in range(_MAX_CHUNKS):
        @pl.when(c < nch_ref[e])
        def _chunk(c=c):
            row = pl.multiple_of(base + c * _CHUNK, 8)
            xc = xs_ref[pl.ds(row, _CHUNK), :]
            gu = jax.lax.dot_general(
                xc, w13, (((1,), (1,)), ((), ())),
                preferred_element_type=jnp.float32)
            gate = gu[:, :inter]
            up = gu[:, inter:]
            h = (gate * jax.nn.sigmoid(gate) * up).astype(jnp.bfloat16)
            o = jax.lax.dot_general(
                h, w2, (((1,), (1,)), ((), ())),
                preferred_element_type=jnp.float32)
            os_ref[pl.ds(row, _CHUNK), :] = o

    @pl.when(e == nexp - 1)
    def _combine():
        out_ref[...] = jax.lax.dot_general(
            p_ref[...], os_ref[...], (((1,), (0,)), ((), ())),
            preferred_element_type=jnp.float32,
            precision=jax.lax.Precision.HIGHEST)


def kernel(hidden_states, topk_ids, topk_weights, w13, w2):
    tokens, hidden = hidden_states.shape
    num_experts, two_inter, _ = w13.shape
    inter = w2.shape[2]
    topk = topk_ids.shape[1]
    nslots = tokens * topk

    ids32 = topk_ids.astype(jnp.int32)
    wts = topk_weights.astype(jnp.float32)
    x16 = hidden_states.astype(jnp.bfloat16)

    eid = ids32.reshape(-1)
    wtv = wts.reshape(-1)
    tokv = (jnp.arange(nslots, dtype=jnp.int32) // topk)
    oh = (eid[:, None] == jnp.arange(num_experts, dtype=jnp.int32)[None, :])
    ohi = oh.astype(jnp.int32)
    cnt = ohi.sum(axis=0)
    pad8 = ((cnt + 7) // 8) * 8
    start = (jnp.cumsum(pad8) - pad8).astype(jnp.int32)
    nch = ((cnt + _CHUNK - 1) // _CHUNK).astype(jnp.int32)
    rank = jnp.sum((jnp.cumsum(ohi, axis=0) - 1) * ohi, axis=1)
    pos = jnp.sum(ohi * start[None, :], axis=1) + rank
    sorted_tok = jnp.zeros(_NP, jnp.int32).at[pos].set(tokv)
    sorted_wt = jnp.zeros(_NP, jnp.float32).at[pos].set(wtv)
    gmat = (sorted_tok[:, None] == jnp.arange(tokens)[None, :]
            ).astype(jnp.bfloat16)
    pmat = ((jnp.arange(tokens)[:, None] == sorted_tok[None, :])
            .astype(jnp.float32) * sorted_wt[None, :])

    out = pl.pallas_call(
        _moe_body,
        grid_spec=pltpu.PrefetchScalarGridSpec(
            num_scalar_prefetch=2,
            grid=(num_experts,),
            in_specs=[
                pl.BlockSpec((_NP, tokens), lambda e, *_: (0, 0)),
                pl.BlockSpec((tokens, _NP), lambda e, *_: (0, 0)),
                pl.BlockSpec((tokens, hidden), lambda e, *_: (0, 0)),
                pl.BlockSpec((1, two_inter, hidden), lambda e, *_: (e, 0, 0)),
                pl.BlockSpec((1, hidden, inter), lambda e, *_: (e, 0, 0)),
            ],
            out_specs=pl.BlockSpec((tokens, hidden), lambda e, *_: (0, 0)),
            scratch_shapes=[
                pltpu.VMEM((_NP, hidden), jnp.bfloat16),
                pltpu.VMEM((_NP, hidden), jnp.float32),
            ],
        ),
        out_shape=jax.ShapeDtypeStruct((tokens, hidden), jnp.float32),
    )(start, nch, gmat, pmat, x16, w13, w2)
    return out
